# Initial kernel scaffold; baseline (speedup 1.0000x reference)
import jax
import jax.numpy as jnp
from jax.experimental import pallas as pl


def kernel(x, block, W1, b1, W2, b2):
    raise NotImplementedError("write your pallas kernel here")


# speedup vs baseline: 13.8195x; 13.8195x over previous
import functools

import jax
import jax.numpy as jnp
from jax import lax
from jax.experimental import pallas as pl
from jax.experimental.pallas import tpu as pltpu
from jax.experimental.pallas import tpu_sc as plsc

_NC = 2
_NS = 16


def _pad_rows(a, np_rows):
    return jnp.pad(a, ((0, np_rows - a.shape[0]), (0, 0)))


def _make_agg(NP, D, E):
    NW = _NC * _NS
    assert E % NW == 0
    epw = E // NW
    K = 80
    assert epw % K == 0 and K % 8 == 0
    C = epw // K
    assert NP % (_NS * 8) == 0
    rpt = NP // _NS
    mesh = plsc.VectorSubcoreMesh(core_axis_name="c", subcore_axis_name="s")

    @functools.partial(
        pl.kernel,
        mesh=mesh,
        out_type=jax.ShapeDtypeStruct((_NC * NP, D), jnp.float32),
        scratch_types=[
            pltpu.VMEM((K,), jnp.int32),
            pltpu.VMEM((K,), jnp.int32),
            pltpu.VMEM((K, D), jnp.float32),
            pltpu.VMEM_SHARED((NP, D), jnp.float32),
            pltpu.SemaphoreType.DMA,
        ],
    )
    def agg(table, srcv, dstv, out, sidx, didx, rows, acc, sem):
        c = lax.axis_index("c")
        s = lax.axis_index("s")
        pltpu.sync_copy(table.at[pl.ds(s * rpt, rpt)], acc.at[pl.ds(s * rpt, rpt)])
        plsc.subcore_barrier()
        base = (c * _NS + s) * epw

        def body(j, carry):
            off = base + j * K
            pltpu.sync_copy(srcv.at[pl.ds(off, K)], sidx)
            pltpu.sync_copy(dstv.at[pl.ds(off, K)], didx)
            pltpu.async_copy(table.at[sidx], rows, sem).wait()
            pltpu.sync_copy(rows, acc.at[didx], add=True)
            return carry

        lax.fori_loop(0, C, body, 0)
        plsc.subcore_barrier()
        pltpu.sync_copy(acc.at[pl.ds(s * rpt, rpt)],
                        out.at[pl.ds(c * NP + s * rpt, rpt)])

    return agg


def _make_deg(NP, D, E):
    NW = _NC * _NS
    assert E % NW == 0
    epw = E // NW
    K = 80
    assert epw % K == 0 and K % 8 == 0
    C = epw // K
    assert NP % (_NS * 8) == 0
    rpt = NP // _NS
    mesh = plsc.VectorSubcoreMesh(core_axis_name="c", subcore_axis_name="s")

    @functools.partial(
        pl.kernel,
        mesh=mesh,
        out_type=jax.ShapeDtypeStruct((_NC * NP, D), jnp.float32),
        scratch_types=[
            pltpu.VMEM((K,), jnp.int32),
            pltpu.VMEM((K, D), jnp.float32),
            pltpu.VMEM_SHARED((NP, D), jnp.float32),
            pltpu.SemaphoreType.DMA,
        ],
    )
    def deg(table, dstv, out, didx, rows, acc, sem):
        c = lax.axis_index("c")
        s = lax.axis_index("s")
        pltpu.sync_copy(table.at[pl.ds(s * rpt, rpt)], acc.at[pl.ds(s * rpt, rpt)])
        pltpu.sync_copy(table.at[pl.ds(0, K)], rows)
        plsc.subcore_barrier()
        base = (c * _NS + s) * epw

        def body(j, carry):
            pltpu.sync_copy(dstv.at[pl.ds(base + j * K, K)], didx)
            pltpu.sync_copy(rows, acc.at[didx], add=True)
            return carry

        lax.fori_loop(0, C, body, 0)
        plsc.subcore_barrier()
        pltpu.sync_copy(acc.at[pl.ds(s * rpt, rpt)],
                        out.at[pl.ds(c * NP + s * rpt, rpt)])

    return deg


def _prep_body(d0, d1, x, g, dis):
    deg = d0[...] + d1[...] - 1.0
    r = lax.rsqrt(deg)
    dis[...] = r
    g[...] = x[...] * r[:, 0:1]


def _mid_body(a0, a1, g, dis, w1, b1, w2, q):
    r = dis[...][:, 0:1]
    m = (a0[...] + a1[...] - g[...]) * r
    h = jnp.maximum(jnp.dot(m, w1[...], preferred_element_type=jnp.float32)
                    + b1[...], 0.0)
    p = jnp.dot(h, w2[...], preferred_element_type=jnp.float32)
    q[...] = p * r


def _final_body(a0, a1, q, dis, b2, o):
    z = (a0[...] + a1[...] - q[...]) * dis[...][:, 0:1] + b2[...]
    zmax = jnp.max(z, axis=1, keepdims=True)
    e = jnp.exp(z - zmax)
    o[...] = z - zmax - jnp.log(jnp.sum(e, axis=1, keepdims=True))


def kernel(x, block, W1, b1, W2, b2):
    N, F_IN = x.shape
    F_HID = W1.shape[1]
    F_OUT = W2.shape[1]
    E = block.shape[1]
    src = block[0].astype(jnp.int32)
    dst = block[1].astype(jnp.int32)
    NP = -(-N // (_NS * 8)) * (_NS * 8)

    B = 1000
    assert N % B == 0
    grid = (N // B,)
    row = lambda i: (i, 0)
    full = lambda i: (0, 0)

    ones8 = jnp.ones((NP, 8), jnp.float32)
    degp = _make_deg(NP, 8, E)(ones8, dst)

    g, dis = pl.pallas_call(
        _prep_body,
        grid=grid,
        in_specs=[pl.BlockSpec((B, 8), row), pl.BlockSpec((B, 8), row),
                  pl.BlockSpec((B, F_IN), row)],
        out_specs=[pl.BlockSpec((B, F_IN), row), pl.BlockSpec((B, 8), row)],
        out_shape=[jax.ShapeDtypeStruct((N, F_IN), jnp.float32),
                   jax.ShapeDtypeStruct((N, 8), jnp.float32)],
    )(degp[:N], degp[NP:NP + N], x)

    a = _make_agg(NP, F_IN, E)(_pad_rows(g, NP), src, dst)

    q = pl.pallas_call(
        _mid_body,
        grid=grid,
        in_specs=[pl.BlockSpec((B, F_IN), row), pl.BlockSpec((B, F_IN), row),
                  pl.BlockSpec((B, F_IN), row), pl.BlockSpec((B, 8), row),
                  pl.BlockSpec((F_IN, F_HID), full),
                  pl.BlockSpec((1, F_HID), full),
                  pl.BlockSpec((F_HID, F_OUT), full)],
        out_specs=pl.BlockSpec((B, F_OUT), row),
        out_shape=jax.ShapeDtypeStruct((N, F_OUT), jnp.float32),
    )(a[:N], a[NP:NP + N], g, dis, W1, b1.reshape(1, F_HID), W2)

    qp = jnp.pad(q, ((0, NP - N), (0, F_IN - F_OUT)))
    a2 = _make_agg(NP, F_IN, E)(qp, src, dst)

    out = pl.pallas_call(
        _final_body,
        grid=grid,
        in_specs=[pl.BlockSpec((B, F_OUT), row), pl.BlockSpec((B, F_OUT), row),
                  pl.BlockSpec((B, F_OUT), row), pl.BlockSpec((B, 8), row),
                  pl.BlockSpec((1, F_OUT), full)],
        out_specs=pl.BlockSpec((B, F_OUT), row),
        out_shape=jax.ShapeDtypeStruct((N, F_OUT), jnp.float32),
    )(a2[:N, :F_OUT], a2[NP:NP + N, :F_OUT], q, dis, b2.reshape(1, F_OUT))

    return out
